# Initial kernel scaffold; baseline (speedup 1.0000x reference)
import functools

import jax
import jax.numpy as jnp
from jax import lax
from jax.experimental import pallas as pl
from jax.experimental.pallas import tpu as pltpu
from jax.experimental.pallas import tpu_sc as plsc

N = 10000
E = 320000
D = 128

NC = 2
NS = 16
NW = NC * NS
EW = E // NW
CHUNK = 80
NCHUNK = EW // CHUNK
ROWS_PER_TILE = N // NS
DEG_LANES = 16

_SC_MESH = dict(core_axis_name="c", subcore_axis_name="s")


@functools.partial(
    pl.kernel,
    out_type=jax.ShapeDtypeStruct((NC, N, DEG_LANES), jnp.float32),
    mesh=plsc.VectorSubcoreMesh(**_SC_MESH),
    scratch_types=[
        pltpu.VMEM((NCHUNK, CHUNK), jnp.int32),
        pltpu.VMEM((CHUNK, DEG_LANES), jnp.float32),
        pltpu.VMEM((ROWS_PER_TILE, DEG_LANES), jnp.float32),
        pltpu.VMEM_SHARED((N, DEG_LANES), jnp.float32),
    ],
)
def _sc_degree(row_hbm, out_hbm, idx_v, ones_v, zeros_v, acc_sh):
    c = lax.axis_index("c")
    s = lax.axis_index("s")

    def fill_ones(i, carry):
        ones_v[i] = jnp.ones((DEG_LANES,), jnp.float32)
        return carry

    lax.fori_loop(0, CHUNK, fill_ones, 0)

    def fill_zeros(i, carry):
        zeros_v[i] = jnp.zeros((DEG_LANES,), jnp.float32)
        return carry

    lax.fori_loop(0, ROWS_PER_TILE, fill_zeros, 0)
    pltpu.sync_copy(zeros_v, acc_sh.at[pl.ds(s * ROWS_PER_TILE, ROWS_PER_TILE)])
    plsc.subcore_barrier()

    pltpu.sync_copy(row_hbm.at[c, s], idx_v)

    def body(j, carry):
        pltpu.sync_copy(ones_v, acc_sh.at[idx_v.at[j]], add=True)
        return carry

    lax.fori_loop(0, NCHUNK, body, 0)
    plsc.subcore_barrier()
    pltpu.sync_copy(
        acc_sh.at[pl.ds(s * ROWS_PER_TILE, ROWS_PER_TILE)],
        out_hbm.at[c, pl.ds(s * ROWS_PER_TILE, ROWS_PER_TILE)],
    )


@functools.partial(
    pl.kernel,
    out_type=jax.ShapeDtypeStruct((NC, N, D), jnp.float32),
    mesh=plsc.VectorSubcoreMesh(**_SC_MESH),
    scratch_types=[
        pltpu.VMEM((NCHUNK, CHUNK), jnp.int32),
        pltpu.VMEM((NCHUNK, CHUNK), jnp.int32),
        pltpu.VMEM((CHUNK, D), jnp.float32),
        pltpu.VMEM((ROWS_PER_TILE, D), jnp.float32),
        pltpu.VMEM_SHARED((N, D), jnp.float32),
        pltpu.SemaphoreType.DMA,
    ],
)
def _sc_scatter(u_hbm, row_hbm, col_hbm, out_hbm,
                row_v, col_v, buf_a, zeros_v, acc_sh, sem_a):
    c = lax.axis_index("c")
    s = lax.axis_index("s")

    def fill_zeros(i, carry):
        r = i // (D // 16)
        k = i % (D // 16)
        zeros_v[r, pl.ds(k * 16, 16)] = jnp.zeros((16,), jnp.float32)
        return carry

    lax.fori_loop(0, ROWS_PER_TILE * (D // 16), fill_zeros, 0)
    pltpu.sync_copy(zeros_v, acc_sh.at[pl.ds(s * ROWS_PER_TILE, ROWS_PER_TILE)])
    plsc.subcore_barrier()

    pltpu.sync_copy(row_hbm.at[c, s], row_v)
    pltpu.sync_copy(col_hbm.at[c, s], col_v)

    def body(j, carry):
        pltpu.async_copy(u_hbm.at[row_v.at[j]], buf_a, sem_a).wait()
        pltpu.sync_copy(buf_a, acc_sh.at[col_v.at[j]], add=True)
        return carry

    lax.fori_loop(0, NCHUNK, body, 0)

    plsc.subcore_barrier()
    pltpu.sync_copy(
        acc_sh.at[pl.ds(s * ROWS_PER_TILE, ROWS_PER_TILE)],
        out_hbm.at[c, pl.ds(s * ROWS_PER_TILE, ROWS_PER_TILE)],
    )


BLK = 1000


def _dis_from_deg(deg_ref):
    deg = deg_ref[0, :, 0:1] + deg_ref[1, :, 0:1] + 1.0
    return lax.rsqrt(deg)


def _tc_encoder_body(x_ref, deg_ref, w1_ref, b1_ref, w2_ref, b2_ref,
                     wc1_ref, bc1_ref, u1_ref, hl1_ref):
    t = jnp.maximum(jnp.dot(x_ref[...], w1_ref[...],
                            preferred_element_type=jnp.float32) + b1_ref[...], 0.0)
    h = jnp.dot(t, w2_ref[...], preferred_element_type=jnp.float32) + b2_ref[...]
    hl1 = jnp.dot(h, wc1_ref[...], preferred_element_type=jnp.float32) + bc1_ref[...]
    dis = _dis_from_deg(deg_ref)
    hl1_ref[...] = hl1
    u1_ref[...] = dis * hl1


def _tc_mid_body(sp_ref, deg_ref, hl1_ref, wc2_ref, bc2_ref, u2_ref, hl2_ref):
    dis = _dis_from_deg(deg_ref)
    ssum = sp_ref[0] + sp_ref[1]
    out1 = jnp.maximum(dis * ssum + dis * dis * hl1_ref[...], 0.0)
    hl2 = jnp.dot(out1, wc2_ref[...], preferred_element_type=jnp.float32) + bc2_ref[...]
    hl2_ref[...] = hl2
    u2_ref[...] = dis * hl2


def _tc_final_body(sp_ref, deg_ref, hl2_ref, ao1_ref, gsum_ref, ns_ref):
    dis = _dis_from_deg(deg_ref)
    ssum = sp_ref[0] + sp_ref[1]
    out2 = jnp.maximum(dis * ssum + dis * dis * hl2_ref[...], 0.0)

    @pl.when(pl.program_id(0) == 0)
    def _():
        gsum_ref[...] = jnp.zeros_like(gsum_ref)

    gsum_ref[...] += jnp.sum(out2, axis=0, keepdims=True)
    ns_ref[...] = jnp.dot(out2, ao1_ref[...], preferred_element_type=jnp.float32)


def _tc_heads_body(gsum_ref, ns_ref, gew_ref, geb_ref, ao1_ref, ao2_ref,
                   aob_ref, vow_ref, vob_ref,
                   scores_ref, end_ref, sv_ref):
    g = gsum_ref[...] * (1.0 / N)
    g2 = jnp.maximum(jnp.dot(g, gew_ref[...],
                             preferred_element_type=jnp.float32) + geb_ref[...], 0.0)
    c2s = jnp.dot(g2, ao2_ref[...], preferred_element_type=jnp.float32) + aob_ref[...]
    end_ref[...] = jnp.dot(g2, ao1_ref[...], preferred_element_type=jnp.float32) + c2s
    sv_ref[...] = jnp.dot(g2, vow_ref[...], preferred_element_type=jnp.float32) + vob_ref[...]
    scores_ref[...] = ns_ref[...] + c2s


def _full_spec(shape):
    return pl.BlockSpec(shape, lambda i, _s=len(shape): (0,) * _s)


_ROW_SPEC = pl.BlockSpec((BLK, D), lambda i: (i, 0))
_DEG_SPEC = pl.BlockSpec((NC, BLK, DEG_LANES), lambda i: (0, i, 0))
_SP_SPEC = pl.BlockSpec((NC, BLK, D), lambda i: (0, i, 0))
_W_SPEC = _full_spec((D, D))
_B_SPEC = _full_spec((1, D))


def kernel(x, edge_index, ne1_W, ne1_b, ne2_W, ne2_b, c1_W, c1_b, c2_W, c2_b,
           ge_W, ge_b, ao_W, ao_b, vo_W, vo_b):
    f32 = jnp.float32
    row = edge_index[0].reshape(NC, NS, NCHUNK, CHUNK)
    col = edge_index[1].reshape(NC, NS, NCHUNK, CHUNK)

    w1t, w2t, wc1t, wc2t, gewt = (w.T for w in (ne1_W, ne2_W, c1_W, c2_W, ge_W))
    b1, b2, bc1, bc2, geb = (b.reshape(1, D) for b in (ne1_b, ne2_b, c1_b, c2_b, ge_b))
    ao1t = ao_W[:, :D].T
    ao2t = ao_W[:, D:].T
    aob = ao_b.reshape(1, 1)
    vowt = vo_W.T
    vob = vo_b.reshape(1, 1)

    degp = _sc_degree(row)

    grid = N // BLK
    u1, hl1 = pl.pallas_call(
        _tc_encoder_body,
        grid=(grid,),
        in_specs=[_ROW_SPEC, _DEG_SPEC] + [_W_SPEC, _B_SPEC] * 3,
        out_specs=[_ROW_SPEC, _ROW_SPEC],
        out_shape=[jax.ShapeDtypeStruct((N, D), f32)] * 2,
    )(x, degp, w1t, b1, w2t, b2, wc1t, bc1)

    s1p = _sc_scatter(u1, row, col)

    u2, hl2 = pl.pallas_call(
        _tc_mid_body,
        grid=(grid,),
        in_specs=[_SP_SPEC, _DEG_SPEC, _ROW_SPEC, _W_SPEC, _B_SPEC],
        out_specs=[_ROW_SPEC, _ROW_SPEC],
        out_shape=[jax.ShapeDtypeStruct((N, D), f32)] * 2,
    )(s1p, degp, hl1, wc2t, bc2)

    s2p = _sc_scatter(u2, row, col)

    gsum, ns = pl.pallas_call(
        _tc_final_body,
        grid=(grid,),
        in_specs=[_SP_SPEC, _DEG_SPEC, _ROW_SPEC, _full_spec((D, 1))],
        out_specs=[pl.BlockSpec((1, D), lambda i: (0, 0)),
                   pl.BlockSpec((BLK, 1), lambda i: (i, 0))],
        out_shape=[jax.ShapeDtypeStruct((1, D), f32),
                   jax.ShapeDtypeStruct((N, 1), f32)],
    )(s2p, degp, hl2, ao1t)

    scores, end, sv = pl.pallas_call(
        _tc_heads_body,
        grid=(1,),
        in_specs=[_full_spec((1, D)), _full_spec((N, 1)), _full_spec((D, D)),
                  _full_spec((1, D)), _full_spec((D, 1)), _full_spec((D, 1)),
                  _full_spec((1, 1)), _full_spec((D, 1)), _full_spec((1, 1))],
        out_specs=[_full_spec((N, 1)), _full_spec((1, 1)), _full_spec((1, 1))],
        out_shape=[jax.ShapeDtypeStruct((N, 1), f32),
                   jax.ShapeDtypeStruct((1, 1), f32),
                   jax.ShapeDtypeStruct((1, 1), f32)],
    )(gsum, ns, gewt, geb, ao1t, ao2t, aob, vowt, vob)

    action_scores = jnp.concatenate([scores.reshape(-1), end.reshape(-1)])
    return (action_scores, sv)


# speedup vs baseline: 18.9925x; 18.9925x over previous
import functools

import jax
import jax.numpy as jnp
from jax import lax
from jax.experimental import pallas as pl
from jax.experimental.pallas import tpu as pltpu
from jax.experimental.pallas import tpu_sc as plsc

N = 10000
E = 320000
D = 128

NC = 2
NS = 16
NW = NC * NS
EW = E // NW
CHUNK = 80
NCHUNK = EW // CHUNK
NPAD = 10240
ROWS_PER_TILE = NPAD // NS
ZROWS = 128
DEG_LANES = 16

_SC_MESH = dict(core_axis_name="c", subcore_axis_name="s")


@functools.partial(
    pl.kernel,
    out_type=jax.ShapeDtypeStruct((NC, NPAD, DEG_LANES), jnp.float32),
    mesh=plsc.VectorSubcoreMesh(**_SC_MESH),
    scratch_types=[
        pltpu.VMEM((NCHUNK, CHUNK), jnp.int32),
        pltpu.VMEM((CHUNK, DEG_LANES), jnp.float32),
        pltpu.VMEM((ROWS_PER_TILE // CHUNK, CHUNK), jnp.int32),
        pltpu.VMEM((CHUNK, DEG_LANES), jnp.float32),
        pltpu.VMEM_SHARED((NPAD, DEG_LANES), jnp.float32),
    ],
)
def _sc_degree(row_hbm, out_hbm, idx_v, ones_v, ridx_v, stage_v, acc_sh):
    c = lax.axis_index("c")
    s = lax.axis_index("s")

    def fill_ones(i, carry):
        ones_v[i] = jnp.ones((DEG_LANES,), jnp.float32)
        return carry

    lax.fori_loop(0, CHUNK, fill_ones, 0)

    def fill_stage(i, carry):
        stage_v[i] = jnp.zeros((DEG_LANES,), jnp.float32)
        return carry

    lax.fori_loop(0, CHUNK, fill_stage, 0)

    for z in range(ROWS_PER_TILE // CHUNK):
        for q in range(CHUNK // 16):
            base = s * ROWS_PER_TILE + z * CHUNK + q * 16
            ridx_v[z, pl.ds(q * 16, 16)] = base + lax.iota(jnp.int32, 16)

    for z in range(ROWS_PER_TILE // CHUNK):
        pltpu.sync_copy(stage_v, acc_sh.at[ridx_v.at[z]])
    plsc.subcore_barrier()

    pltpu.sync_copy(row_hbm.at[c, s], idx_v)

    def body(j, carry):
        pltpu.sync_copy(ones_v, acc_sh.at[idx_v.at[j]], add=True)
        return carry

    lax.fori_loop(0, NCHUNK, body, 0)
    plsc.subcore_barrier()

    for z in range(ROWS_PER_TILE // CHUNK):
        pltpu.sync_copy(acc_sh.at[ridx_v.at[z]], stage_v)
        pltpu.sync_copy(
            stage_v,
            out_hbm.at[c, pl.ds(s * ROWS_PER_TILE + z * CHUNK, CHUNK)])


@functools.partial(
    pl.kernel,
    out_type=jax.ShapeDtypeStruct((NC, NPAD, D), jnp.float32),
    mesh=plsc.VectorSubcoreMesh(**_SC_MESH),
    scratch_types=[
        pltpu.VMEM((NCHUNK, CHUNK), jnp.int32),
        pltpu.VMEM((NCHUNK, CHUNK), jnp.int32),
        pltpu.VMEM((CHUNK, D), jnp.float32),
        pltpu.VMEM((ROWS_PER_TILE // CHUNK, CHUNK), jnp.int32),
        pltpu.VMEM_SHARED((NPAD, D), jnp.float32),
        pltpu.SemaphoreType.DMA,
    ],
)
def _sc_scatter(u_hbm, row_hbm, col_hbm, out_hbm,
                row_v, col_v, buf_a, ridx_v, acc_sh, sem_a):
    c = lax.axis_index("c")
    s = lax.axis_index("s")

    def fill_zeros(i, carry):
        r = i // (D // 16)
        k = i % (D // 16)
        buf_a[r, pl.ds(k * 16, 16)] = jnp.zeros((16,), jnp.float32)
        return carry

    lax.fori_loop(0, CHUNK * (D // 16), fill_zeros, 0)

    for z in range(ROWS_PER_TILE // CHUNK):
        for q in range(CHUNK // 16):
            base = s * ROWS_PER_TILE + z * CHUNK + q * 16
            ridx_v[z, pl.ds(q * 16, 16)] = base + lax.iota(jnp.int32, 16)

    for z in range(ROWS_PER_TILE // CHUNK):
        pltpu.sync_copy(buf_a, acc_sh.at[ridx_v.at[z]])
    plsc.subcore_barrier()

    pltpu.sync_copy(row_hbm.at[c, s], row_v)
    pltpu.sync_copy(col_hbm.at[c, s], col_v)

    def body(j, carry):
        pltpu.async_copy(u_hbm.at[row_v.at[j]], buf_a, sem_a).wait()
        pltpu.sync_copy(buf_a, acc_sh.at[col_v.at[j]], add=True)
        return carry

    lax.fori_loop(0, NCHUNK, body, 0)

    plsc.subcore_barrier()
    for z in range(ROWS_PER_TILE // CHUNK):
        base = s * ROWS_PER_TILE + z * CHUNK
        pltpu.sync_copy(acc_sh.at[ridx_v.at[z]], buf_a)
        pltpu.sync_copy(buf_a, out_hbm.at[c, pl.ds(base, CHUNK)])


BLK = 1000


def _dis_from_deg(deg_ref):
    deg = deg_ref[0, :, 0:1] + deg_ref[1, :, 0:1] + 1.0
    return lax.rsqrt(deg)


def _tc_encoder_body(x_ref, deg_ref, w1_ref, b1_ref, w2_ref, b2_ref,
                     wc1_ref, bc1_ref, u1_ref, hl1_ref):
    t = jnp.maximum(jnp.dot(x_ref[...], w1_ref[...],
                            preferred_element_type=jnp.float32) + b1_ref[...], 0.0)
    h = jnp.dot(t, w2_ref[...], preferred_element_type=jnp.float32) + b2_ref[...]
    hl1 = jnp.dot(h, wc1_ref[...], preferred_element_type=jnp.float32) + bc1_ref[...]
    dis = _dis_from_deg(deg_ref)
    hl1_ref[...] = hl1
    u1_ref[...] = dis * hl1


def _tc_mid_body(sp_ref, deg_ref, hl1_ref, wc2_ref, bc2_ref, u2_ref, hl2_ref):
    dis = _dis_from_deg(deg_ref)
    ssum = sp_ref[0] + sp_ref[1]
    out1 = jnp.maximum(dis * ssum + dis * dis * hl1_ref[...], 0.0)
    hl2 = jnp.dot(out1, wc2_ref[...], preferred_element_type=jnp.float32) + bc2_ref[...]
    hl2_ref[...] = hl2
    u2_ref[...] = dis * hl2


def _tc_final_body(sp_ref, deg_ref, hl2_ref, ao1_ref, gsum_ref, ns_ref):
    dis = _dis_from_deg(deg_ref)
    ssum = sp_ref[0] + sp_ref[1]
    out2 = jnp.maximum(dis * ssum + dis * dis * hl2_ref[...], 0.0)

    @pl.when(pl.program_id(0) == 0)
    def _():
        gsum_ref[...] = jnp.zeros_like(gsum_ref)

    gsum_ref[...] += jnp.sum(out2, axis=0, keepdims=True)
    ns_ref[...] = jnp.dot(out2, ao1_ref[...], preferred_element_type=jnp.float32)


def _tc_heads_body(gsum_ref, ns_ref, gew_ref, geb_ref, ao1_ref, ao2_ref,
                   aob_ref, vow_ref, vob_ref,
                   scores_ref, end_ref, sv_ref):
    g = gsum_ref[...] * (1.0 / N)
    g2 = jnp.maximum(jnp.dot(g, gew_ref[...],
                             preferred_element_type=jnp.float32) + geb_ref[...], 0.0)
    c2s = jnp.dot(g2, ao2_ref[...], preferred_element_type=jnp.float32) + aob_ref[...]
    end_ref[...] = jnp.dot(g2, ao1_ref[...], preferred_element_type=jnp.float32) + c2s
    sv_ref[...] = jnp.dot(g2, vow_ref[...], preferred_element_type=jnp.float32) + vob_ref[...]
    scores_ref[...] = ns_ref[...] + c2s


def _full_spec(shape):
    return pl.BlockSpec(shape, lambda i, _s=len(shape): (0,) * _s)


_ROW_SPEC = pl.BlockSpec((BLK, D), lambda i: (i, 0))
_DEG_SPEC = pl.BlockSpec((NC, BLK, DEG_LANES), lambda i: (0, i, 0))
_SP_SPEC = pl.BlockSpec((NC, BLK, D), lambda i: (0, i, 0))
_W_SPEC = _full_spec((D, D))
_B_SPEC = _full_spec((1, D))


def kernel(x, edge_index, ne1_W, ne1_b, ne2_W, ne2_b, c1_W, c1_b, c2_W, c2_b,
           ge_W, ge_b, ao_W, ao_b, vo_W, vo_b):
    f32 = jnp.float32
    row = edge_index[0].reshape(NC, NS, NCHUNK, CHUNK)
    col = edge_index[1].reshape(NC, NS, NCHUNK, CHUNK)

    w1t, w2t, wc1t, wc2t, gewt = (w.T for w in (ne1_W, ne2_W, c1_W, c2_W, ge_W))
    b1, b2, bc1, bc2, geb = (b.reshape(1, D) for b in (ne1_b, ne2_b, c1_b, c2_b, ge_b))
    ao1t = ao_W[:, :D].T
    ao2t = ao_W[:, D:].T
    aob = ao_b.reshape(1, 1)
    vowt = vo_W.T
    vob = vo_b.reshape(1, 1)

    def _xla_degree(row):
        cnt = jnp.zeros((NPAD,), f32).at[row.reshape(-1)].add(1.0)
        d = jnp.broadcast_to(cnt[None, :, None], (1, NPAD, DEG_LANES))
        return jnp.concatenate([d, jnp.zeros((1, NPAD, DEG_LANES), f32)], axis=0)

    def _xla_scatter(u, row, col):
        s = jnp.zeros((NPAD, D), f32).at[col.reshape(-1)].add(u[row.reshape(-1)])
        return jnp.stack([s, jnp.zeros((NPAD, D), f32)])

    degp = _sc_degree(row)
    del _xla_degree

    grid = N // BLK
    u1, hl1 = pl.pallas_call(
        _tc_encoder_body,
        grid=(grid,),
        in_specs=[_ROW_SPEC, _DEG_SPEC] + [_W_SPEC, _B_SPEC] * 3,
        out_specs=[_ROW_SPEC, _ROW_SPEC],
        out_shape=[jax.ShapeDtypeStruct((N, D), f32)] * 2,
    )(x, degp, w1t, b1, w2t, b2, wc1t, bc1)

    s1p = _sc_scatter(u1, row, col)

    u2, hl2 = pl.pallas_call(
        _tc_mid_body,
        grid=(grid,),
        in_specs=[_SP_SPEC, _DEG_SPEC, _ROW_SPEC, _W_SPEC, _B_SPEC],
        out_specs=[_ROW_SPEC, _ROW_SPEC],
        out_shape=[jax.ShapeDtypeStruct((N, D), f32)] * 2,
    )(s1p, degp, hl1, wc2t, bc2)

    s2p = _sc_scatter(u2, row, col)

    gsum, ns = pl.pallas_call(
        _tc_final_body,
        grid=(grid,),
        in_specs=[_SP_SPEC, _DEG_SPEC, _ROW_SPEC, _full_spec((D, 1))],
        out_specs=[pl.BlockSpec((1, D), lambda i: (0, 0)),
                   pl.BlockSpec((BLK, 1), lambda i: (i, 0))],
        out_shape=[jax.ShapeDtypeStruct((1, D), f32),
                   jax.ShapeDtypeStruct((N, 1), f32)],
    )(s2p, degp, hl2, ao1t)

    scores, end, sv = pl.pallas_call(
        _tc_heads_body,
        grid=(1,),
        in_specs=[_full_spec((1, D)), _full_spec((N, 1)), _full_spec((D, D)),
                  _full_spec((1, D)), _full_spec((D, 1)), _full_spec((D, 1)),
                  _full_spec((1, 1)), _full_spec((D, 1)), _full_spec((1, 1))],
        out_specs=[_full_spec((N, 1)), _full_spec((1, 1)), _full_spec((1, 1))],
        out_shape=[jax.ShapeDtypeStruct((N, 1), f32),
                   jax.ShapeDtypeStruct((1, 1), f32),
                   jax.ShapeDtypeStruct((1, 1), f32)],
    )(gsum, ns, gewt, geb, ao1t, ao2t, aob, vowt, vob)

    action_scores = jnp.concatenate([scores.reshape(-1), end.reshape(-1)])
    return (action_scores, sv)
